# baseline (speedup 1.0000x reference)
import functools

import jax
import jax.numpy as jnp
import numpy as np
from jax.experimental import pallas as pl
from jax.experimental.pallas import tpu as pltpu

_EPS_BN = 1e-5
_INV = np.float32(1.0) / np.sqrt(np.float32(1.0 + _EPS_BN))


def _fps_idx(pos, num_samples):
    p = jax.lax.stop_gradient(pos)
    n = p.shape[0]
    idxs0 = jnp.zeros((num_samples,), dtype=jnp.int32)
    dists0 = jnp.full((n,), jnp.inf, dtype=jnp.float32)

    def body(i, state):
        dists, idxs = state
        last = idxs[i - 1]
        d = jnp.sum((p - p[last]) ** 2, axis=1)
        dists = jnp.minimum(dists, d)
        idxs = idxs.at[i].set(jnp.argmax(dists).astype(jnp.int32))
        return (dists, idxs)

    _, idxs = jax.lax.fori_loop(1, num_samples, body, (dists0, idxs0))
    return idxs


def _sa_body(nl, kk, r2s, fouts, *refs):
    h_ref, d2_ref = refs[0], refs[1]
    wrefs = refs[2:-1]
    out_ref = refs[-1]
    h0 = h_ref[...]
    d2col = d2_ref[...]
    bm = d2col.shape[0] // kk
    col = 0
    for bi, r2 in enumerate(r2s):
        h = h0
        base = bi * nl * 4
        for li in range(nl):
            w = wrefs[base + li * 4][...]
            b = wrefs[base + li * 4 + 1][...]
            g = wrefs[base + li * 4 + 2][...]
            be = wrefs[base + li * 4 + 3][...]
            h = jnp.maximum(
                jnp.dot(h, w, preferred_element_type=jnp.float32) + b, 0.0)
            h = g * (h * _INV) + be
        fo = fouts[bi]
        penalty = jnp.where(d2col <= r2, 0.0, -jnp.inf)
        h = h + penalty
        o = jnp.max(h.reshape(bm, kk, fo), axis=1)
        o = jnp.where(jnp.isfinite(o), o, 0.0)
        out_ref[:, col:col + fo] = o
        col += fo


def _sa_conv(h_in, d2k, r_list, conv_params, bm):
    m, kk, f = h_in.shape
    h_flat = h_in.reshape(m * kk, f)
    d2col = d2k.reshape(m * kk, 1)
    nl = len(conv_params[0])
    fouts = tuple(int(layers[-1]["W"].shape[1]) for layers in conv_params)
    r2s = tuple(np.float32(r * r) for r in r_list)
    wargs, wspecs = [], []
    for layers in conv_params:
        for lyr in layers:
            for nm in ("W", "b", "gamma", "beta"):
                a = lyr[nm]
                if a.ndim == 1:
                    a = a.reshape(1, -1)
                wargs.append(a)
                wspecs.append(pl.BlockSpec(a.shape, lambda i: (0, 0)))
    out_f = sum(fouts)
    grid = (m // bm,)
    fn = pl.pallas_call(
        functools.partial(_sa_body, nl, kk, r2s, fouts),
        grid=grid,
        in_specs=[
            pl.BlockSpec((bm * kk, f), lambda i: (i, 0)),
            pl.BlockSpec((bm * kk, 1), lambda i: (i, 0)),
        ] + wspecs,
        out_specs=pl.BlockSpec((bm, out_f), lambda i: (i, 0)),
        out_shape=jax.ShapeDtypeStruct((m, out_f), jnp.float32),
    )
    return fn(h_flat, d2col, *wargs)


def _sa_module(x, pos, ratio, r_list, conv_params, bm, max_nbrs=128):
    n = pos.shape[0]
    m = int(round(ratio * n))
    idx = _fps_idx(pos, m)
    y_pos = pos[idx]
    d2_sg = jax.lax.stop_gradient(
        jnp.sum((y_pos[:, None, :] - pos[None, :, :]) ** 2, axis=-1))
    neg_vals, nbr = jax.lax.top_k(-d2_sg, max_nbrs)
    d2k = -neg_vals
    x_j = x[nbr]
    rel = pos[nbr] - y_pos[:, None, :]
    h_in = jnp.concatenate([x_j, rel], axis=-1)
    return _sa_conv(h_in, d2k, r_list, conv_params, bm), y_pos


def _mlp_body(nl, with_head, *refs):
    h_ref = refs[0]
    wrefs = refs[1:-1]
    out_ref = refs[-1]
    h = h_ref[...]
    for li in range(nl):
        w = wrefs[li * 4][...]
        b = wrefs[li * 4 + 1][...]
        g = wrefs[li * 4 + 2][...]
        be = wrefs[li * 4 + 3][...]
        h = jnp.maximum(
            jnp.dot(h, w, preferred_element_type=jnp.float32) + b, 0.0)
        h = g * (h * _INV) + be
    if with_head:
        base = nl * 4
        w1, b1 = wrefs[base][...], wrefs[base + 1][...]
        w2, b2 = wrefs[base + 2][...], wrefs[base + 3][...]
        w3, b3 = wrefs[base + 4][...], wrefs[base + 5][...]
        h = jnp.maximum(jnp.dot(h, w1, preferred_element_type=jnp.float32) + b1, 0.0)
        h = jnp.maximum(jnp.dot(h, w2, preferred_element_type=jnp.float32) + b2, 0.0)
        h = jnp.dot(h, w3, preferred_element_type=jnp.float32) + b3
        mx = jnp.max(h, axis=-1, keepdims=True)
        sh = h - jax.lax.stop_gradient(mx)
        h = sh - jnp.log(jnp.sum(jnp.exp(sh), axis=-1, keepdims=True))
    out_ref[...] = h


def _mlp_rows(h, layers, br, head=None):
    rows, f = h.shape
    nl = len(layers)
    wargs, wspecs = [], []
    for lyr in layers:
        for nm in ("W", "b", "gamma", "beta"):
            a = lyr[nm]
            if a.ndim == 1:
                a = a.reshape(1, -1)
            wargs.append(a)
            wspecs.append(pl.BlockSpec(a.shape, lambda i: (0, 0)))
    if head is not None:
        for nm in ("W1", "b1", "W2", "b2", "W3", "b3"):
            a = head[nm]
            if a.ndim == 1:
                a = a.reshape(1, -1)
            wargs.append(a)
            wspecs.append(pl.BlockSpec(a.shape, lambda i: (0, 0)))
        out_f = head["W3"].shape[1]
    else:
        out_f = layers[-1]["W"].shape[1]
    fn = pl.pallas_call(
        functools.partial(_mlp_body, nl, head is not None),
        grid=(rows // br,),
        in_specs=[pl.BlockSpec((br, f), lambda i: (i, 0))] + wspecs,
        out_specs=pl.BlockSpec((br, out_f), lambda i: (i, 0)),
        out_shape=jax.ShapeDtypeStruct((rows, out_f), jnp.float32),
    )
    return fn(h, *wargs)


def _knn_interpolate(xf, posc, pos_skip, k=3):
    d2_sg = jax.lax.stop_gradient(
        jnp.sum((pos_skip[:, None, :] - posc[None, :, :]) ** 2, axis=-1))
    _, idx = jax.lax.top_k(-d2_sg, k)
    diff = pos_skip[:, None, :] - posc[idx]
    d2 = jnp.sum(diff ** 2, axis=-1)
    w = 1.0 / jnp.maximum(d2, 1e-16)
    w = w / jnp.sum(w, axis=1, keepdims=True)
    return jnp.sum(xf[idx] * w[:, :, None], axis=1)


def kernel(x, pos, batch, params):
    x1, pos1 = _sa_module(x, pos, 0.25, [0.05, 0.4], params["sa1"], bm=32)
    x2, pos2 = _sa_module(x1, pos1, 0.25, [0.2, 0.8], params["sa2"], bm=32)
    x3, pos3 = _sa_module(x2, pos2, 0.25, [0.4, 1.6], params["sa3"], bm=32)

    xi3 = _knn_interpolate(x3, pos3, pos2)
    f3 = _mlp_rows(jnp.concatenate([xi3, x2], axis=1), params["fp3"], br=256)
    xi2 = _knn_interpolate(f3, pos2, pos1)
    f2 = _mlp_rows(jnp.concatenate([xi2, x1], axis=1), params["fp2"], br=512)
    xi1 = _knn_interpolate(f2, pos1, pos)
    f1 = _mlp_rows(jnp.concatenate([xi1, x], axis=1), params["fp1"], br=1024)

    return _mlp_rows(f1, [], br=1024, head=params["cls"])


# speedup vs baseline: 1.8213x; 1.8213x over previous
import functools

import jax
import jax.numpy as jnp
import numpy as np
from jax.experimental import pallas as pl
from jax.experimental.pallas import tpu as pltpu

_EPS_BN = 1e-5
_INV = np.float32(1.0) / np.sqrt(np.float32(1.0 + _EPS_BN))


def _fps_body(m, px_ref, py_ref, pz_ref, out_ref):
    px = px_ref[...]
    py = py_ref[...]
    pz = pz_ref[...]
    r = px.shape[0]
    row = jax.lax.broadcasted_iota(jnp.int32, (r, 128), 0)
    colv = jax.lax.broadcasted_iota(jnp.int32, (r, 128), 1)
    flat = row * 128 + colv
    out_ref[...] = jnp.zeros(out_ref.shape, jnp.int32)
    dists0 = jnp.full((r, 128), jnp.inf, jnp.float32)

    def body(i, carry):
        dists, last = carry
        sel = flat == last
        lx = jnp.sum(jnp.where(sel, px, 0.0))
        ly = jnp.sum(jnp.where(sel, py, 0.0))
        lz = jnp.sum(jnp.where(sel, pz, 0.0))
        dxx = px - lx
        dyy = py - ly
        dzz = pz - lz
        d = dxx * dxx + dyy * dyy + dzz * dzz
        dists = jnp.minimum(dists, d)
        mx = jnp.max(dists)
        idx = jnp.min(jnp.where(dists == mx, flat, jnp.int32(2 ** 30)))
        out_ref[pl.ds(i, 1), :] = jnp.reshape(idx, (1, 1))
        return (dists, idx)

    jax.lax.fori_loop(1, m, body, (dists0, jnp.int32(0)))


def _fps_idx(pos, num_samples):
    n = pos.shape[0]
    r = n // 128
    px = pos[:, 0].reshape(r, 128)
    py = pos[:, 1].reshape(r, 128)
    pz = pos[:, 2].reshape(r, 128)
    out = pl.pallas_call(
        functools.partial(_fps_body, num_samples),
        out_shape=jax.ShapeDtypeStruct((num_samples, 1), jnp.int32),
    )(px, py, pz)
    return out.reshape(num_samples)


def _sa_body(nl, kk, r2s, fouts, *refs):
    h_ref, d2_ref = refs[0], refs[1]
    wrefs = refs[2:-1]
    out_ref = refs[-1]
    h0 = h_ref[...]
    d2col = d2_ref[...]
    bm = d2col.shape[0] // kk
    col = 0
    for bi, r2 in enumerate(r2s):
        h = h0
        base = bi * nl * 4
        for li in range(nl):
            w = wrefs[base + li * 4][...]
            b = wrefs[base + li * 4 + 1][...]
            g = wrefs[base + li * 4 + 2][...]
            be = wrefs[base + li * 4 + 3][...]
            h = jnp.maximum(
                jnp.dot(h, w, preferred_element_type=jnp.float32) + b, 0.0)
            h = g * (h * _INV) + be
        fo = fouts[bi]
        penalty = jnp.where(d2col <= r2, 0.0, -jnp.inf)
        h = h + penalty
        o = jnp.max(h.reshape(bm, kk, fo), axis=1)
        o = jnp.where(jnp.isfinite(o), o, 0.0)
        out_ref[:, col:col + fo] = o
        col += fo


def _sa_conv(h_in, d2k, r_list, conv_params, bm):
    m, kk, f = h_in.shape
    h_flat = h_in.reshape(m * kk, f)
    d2col = d2k.reshape(m * kk, 1)
    nl = len(conv_params[0])
    fouts = tuple(int(layers[-1]["W"].shape[1]) for layers in conv_params)
    r2s = tuple(np.float32(r * r) for r in r_list)
    wargs, wspecs = [], []
    for layers in conv_params:
        for lyr in layers:
            for nm in ("W", "b", "gamma", "beta"):
                a = lyr[nm]
                if a.ndim == 1:
                    a = a.reshape(1, -1)
                wargs.append(a)
                wspecs.append(pl.BlockSpec(a.shape, lambda i: (0, 0)))
    out_f = sum(fouts)
    grid = (m // bm,)
    fn = pl.pallas_call(
        functools.partial(_sa_body, nl, kk, r2s, fouts),
        grid=grid,
        in_specs=[
            pl.BlockSpec((bm * kk, f), lambda i: (i, 0)),
            pl.BlockSpec((bm * kk, 1), lambda i: (i, 0)),
        ] + wspecs,
        out_specs=pl.BlockSpec((bm, out_f), lambda i: (i, 0)),
        out_shape=jax.ShapeDtypeStruct((m, out_f), jnp.float32),
    )
    return fn(h_flat, d2col, *wargs)


def _sa_module(x, pos, ratio, r_list, conv_params, bm, max_nbrs=128):
    n = pos.shape[0]
    m = int(round(ratio * n))
    idx = _fps_idx(pos, m)
    y_pos = pos[idx]
    d2_sg = jax.lax.stop_gradient(
        jnp.sum((y_pos[:, None, :] - pos[None, :, :]) ** 2, axis=-1))
    neg_vals, nbr = jax.lax.top_k(-d2_sg, max_nbrs)
    d2k = -neg_vals
    x_j = x[nbr]
    rel = pos[nbr] - y_pos[:, None, :]
    h_in = jnp.concatenate([x_j, rel], axis=-1)
    return _sa_conv(h_in, d2k, r_list, conv_params, bm), y_pos


def _mlp_body(nl, with_head, *refs):
    h_ref = refs[0]
    wrefs = refs[1:-1]
    out_ref = refs[-1]
    h = h_ref[...]
    for li in range(nl):
        w = wrefs[li * 4][...]
        b = wrefs[li * 4 + 1][...]
        g = wrefs[li * 4 + 2][...]
        be = wrefs[li * 4 + 3][...]
        h = jnp.maximum(
            jnp.dot(h, w, preferred_element_type=jnp.float32) + b, 0.0)
        h = g * (h * _INV) + be
    if with_head:
        base = nl * 4
        w1, b1 = wrefs[base][...], wrefs[base + 1][...]
        w2, b2 = wrefs[base + 2][...], wrefs[base + 3][...]
        w3, b3 = wrefs[base + 4][...], wrefs[base + 5][...]
        h = jnp.maximum(jnp.dot(h, w1, preferred_element_type=jnp.float32) + b1, 0.0)
        h = jnp.maximum(jnp.dot(h, w2, preferred_element_type=jnp.float32) + b2, 0.0)
        h = jnp.dot(h, w3, preferred_element_type=jnp.float32) + b3
        mx = jnp.max(h, axis=-1, keepdims=True)
        sh = h - jax.lax.stop_gradient(mx)
        h = sh - jnp.log(jnp.sum(jnp.exp(sh), axis=-1, keepdims=True))
    out_ref[...] = h


def _mlp_rows(h, layers, br, head=None):
    rows, f = h.shape
    nl = len(layers)
    wargs, wspecs = [], []
    for lyr in layers:
        for nm in ("W", "b", "gamma", "beta"):
            a = lyr[nm]
            if a.ndim == 1:
                a = a.reshape(1, -1)
            wargs.append(a)
            wspecs.append(pl.BlockSpec(a.shape, lambda i: (0, 0)))
    if head is not None:
        for nm in ("W1", "b1", "W2", "b2", "W3", "b3"):
            a = head[nm]
            if a.ndim == 1:
                a = a.reshape(1, -1)
            wargs.append(a)
            wspecs.append(pl.BlockSpec(a.shape, lambda i: (0, 0)))
        out_f = head["W3"].shape[1]
    else:
        out_f = layers[-1]["W"].shape[1]
    fn = pl.pallas_call(
        functools.partial(_mlp_body, nl, head is not None),
        grid=(rows // br,),
        in_specs=[pl.BlockSpec((br, f), lambda i: (i, 0))] + wspecs,
        out_specs=pl.BlockSpec((br, out_f), lambda i: (i, 0)),
        out_shape=jax.ShapeDtypeStruct((rows, out_f), jnp.float32),
    )
    return fn(h, *wargs)


def _knn_interpolate(xf, posc, pos_skip, k=3):
    d2_sg = jax.lax.stop_gradient(
        jnp.sum((pos_skip[:, None, :] - posc[None, :, :]) ** 2, axis=-1))
    _, idx = jax.lax.top_k(-d2_sg, k)
    diff = pos_skip[:, None, :] - posc[idx]
    d2 = jnp.sum(diff ** 2, axis=-1)
    w = 1.0 / jnp.maximum(d2, 1e-16)
    w = w / jnp.sum(w, axis=1, keepdims=True)
    return jnp.sum(xf[idx] * w[:, :, None], axis=1)


def kernel(x, pos, batch, params):
    x1, pos1 = _sa_module(x, pos, 0.25, [0.05, 0.4], params["sa1"], bm=32)
    x2, pos2 = _sa_module(x1, pos1, 0.25, [0.2, 0.8], params["sa2"], bm=32)
    x3, pos3 = _sa_module(x2, pos2, 0.25, [0.4, 1.6], params["sa3"], bm=32)

    xi3 = _knn_interpolate(x3, pos3, pos2)
    f3 = _mlp_rows(jnp.concatenate([xi3, x2], axis=1), params["fp3"], br=256)
    xi2 = _knn_interpolate(f3, pos2, pos1)
    f2 = _mlp_rows(jnp.concatenate([xi2, x1], axis=1), params["fp2"], br=512)
    xi1 = _knn_interpolate(f2, pos1, pos)
    f1 = _mlp_rows(jnp.concatenate([xi1, x], axis=1), params["fp1"], br=1024)

    return _mlp_rows(f1, [], br=1024, head=params["cls"])


# speedup vs baseline: 1.8338x; 1.0069x over previous
import functools

import jax
import jax.numpy as jnp
import numpy as np
from jax.experimental import pallas as pl
from jax.experimental.pallas import tpu as pltpu

_EPS_BN = 1e-5
_INV = np.float32(1.0) / np.sqrt(np.float32(1.0 + _EPS_BN))


def _fps_body(m, px_ref, py_ref, pz_ref, out_ref):
    px = px_ref[...]
    py = py_ref[...]
    pz = pz_ref[...]
    r = px.shape[0]
    row = jax.lax.broadcasted_iota(jnp.int32, (r, 128), 0)
    colv = jax.lax.broadcasted_iota(jnp.int32, (r, 128), 1)
    flat = row * 128 + colv
    out_ref[...] = jnp.zeros(out_ref.shape, jnp.int32)
    dists0 = jnp.full((r, 128), jnp.inf, jnp.float32)

    def body(i, carry):
        dists, last = carry
        sel = flat == last
        lx = jnp.sum(jnp.where(sel, px, 0.0))
        ly = jnp.sum(jnp.where(sel, py, 0.0))
        lz = jnp.sum(jnp.where(sel, pz, 0.0))
        dxx = px - lx
        dyy = py - ly
        dzz = pz - lz
        d = dxx * dxx + dyy * dyy + dzz * dzz
        dists = jnp.minimum(dists, d)
        mx = jnp.max(dists)
        idx = jnp.min(jnp.where(dists == mx, flat, jnp.int32(2 ** 30)))
        out_ref[pl.ds(i, 1), :] = jnp.reshape(idx, (1, 1))
        return (dists, idx)

    jax.lax.fori_loop(1, m, body, (dists0, jnp.int32(0)))


def _fps_idx(pos, num_samples):
    n = pos.shape[0]
    r = n // 128
    px = pos[:, 0].reshape(r, 128)
    py = pos[:, 1].reshape(r, 128)
    pz = pos[:, 2].reshape(r, 128)
    out = pl.pallas_call(
        functools.partial(_fps_body, num_samples),
        out_shape=jax.ShapeDtypeStruct((num_samples, 1), jnp.int32),
    )(px, py, pz)
    return out.reshape(num_samples)


def _sa_body(nl, kk, r2s, fouts, *refs):
    h_ref, d2_ref = refs[0], refs[1]
    wrefs = refs[2:-1]
    out_ref = refs[-1]
    h0 = h_ref[...]
    d2col = d2_ref[...]
    bm = d2col.shape[0] // kk
    col = 0
    for bi, r2 in enumerate(r2s):
        h = h0
        base = bi * nl * 4
        for li in range(nl):
            w = wrefs[base + li * 4][...]
            b = wrefs[base + li * 4 + 1][...]
            g = wrefs[base + li * 4 + 2][...]
            be = wrefs[base + li * 4 + 3][...]
            h = jnp.maximum(
                jnp.dot(h, w, preferred_element_type=jnp.float32) + b, 0.0)
            h = g * (h * _INV) + be
        fo = fouts[bi]
        penalty = jnp.where(d2col <= r2, 0.0, -jnp.inf)
        h = h + penalty
        o = jnp.max(h.reshape(bm, kk, fo), axis=1)
        o = jnp.where(jnp.isfinite(o), o, 0.0)
        out_ref[:, col:col + fo] = o
        col += fo


def _sa_conv(h_in, d2k, r_list, conv_params, bm):
    m, kk, f = h_in.shape
    h_flat = h_in.reshape(m * kk, f)
    d2col = d2k.reshape(m * kk, 1)
    nl = len(conv_params[0])
    fouts = tuple(int(layers[-1]["W"].shape[1]) for layers in conv_params)
    r2s = tuple(np.float32(r * r) for r in r_list)
    wargs, wspecs = [], []
    for layers in conv_params:
        for lyr in layers:
            for nm in ("W", "b", "gamma", "beta"):
                a = lyr[nm]
                if a.ndim == 1:
                    a = a.reshape(1, -1)
                wargs.append(a)
                wspecs.append(pl.BlockSpec(a.shape, lambda i: (0, 0)))
    out_f = sum(fouts)
    grid = (m // bm,)
    fn = pl.pallas_call(
        functools.partial(_sa_body, nl, kk, r2s, fouts),
        grid=grid,
        in_specs=[
            pl.BlockSpec((bm * kk, f), lambda i: (i, 0)),
            pl.BlockSpec((bm * kk, 1), lambda i: (i, 0)),
        ] + wspecs,
        out_specs=pl.BlockSpec((bm, out_f), lambda i: (i, 0)),
        out_shape=jax.ShapeDtypeStruct((m, out_f), jnp.float32),
    )
    return fn(h_flat, d2col, *wargs)


def _sa_module(x, pos, ratio, r_list, conv_params, bm, max_nbrs=128):
    n = pos.shape[0]
    m = int(round(ratio * n))
    idx = _fps_idx(pos, m)
    y_pos = pos[idx]
    d2_sg = jax.lax.stop_gradient(
        jnp.sum((y_pos[:, None, :] - pos[None, :, :]) ** 2, axis=-1))
    neg_vals, nbr = jax.lax.approx_max_k(-d2_sg, max_nbrs, recall_target=1.0)
    d2k = -neg_vals
    x_j = x[nbr]
    rel = pos[nbr] - y_pos[:, None, :]
    h_in = jnp.concatenate([x_j, rel], axis=-1)
    return _sa_conv(h_in, d2k, r_list, conv_params, bm), y_pos


def _mlp_body(nl, with_head, *refs):
    h_ref = refs[0]
    wrefs = refs[1:-1]
    out_ref = refs[-1]
    h = h_ref[...]
    for li in range(nl):
        w = wrefs[li * 4][...]
        b = wrefs[li * 4 + 1][...]
        g = wrefs[li * 4 + 2][...]
        be = wrefs[li * 4 + 3][...]
        h = jnp.maximum(
            jnp.dot(h, w, preferred_element_type=jnp.float32) + b, 0.0)
        h = g * (h * _INV) + be
    if with_head:
        base = nl * 4
        w1, b1 = wrefs[base][...], wrefs[base + 1][...]
        w2, b2 = wrefs[base + 2][...], wrefs[base + 3][...]
        w3, b3 = wrefs[base + 4][...], wrefs[base + 5][...]
        h = jnp.maximum(jnp.dot(h, w1, preferred_element_type=jnp.float32) + b1, 0.0)
        h = jnp.maximum(jnp.dot(h, w2, preferred_element_type=jnp.float32) + b2, 0.0)
        h = jnp.dot(h, w3, preferred_element_type=jnp.float32) + b3
        mx = jnp.max(h, axis=-1, keepdims=True)
        sh = h - jax.lax.stop_gradient(mx)
        h = sh - jnp.log(jnp.sum(jnp.exp(sh), axis=-1, keepdims=True))
    out_ref[...] = h


def _mlp_rows(h, layers, br, head=None):
    rows, f = h.shape
    nl = len(layers)
    wargs, wspecs = [], []
    for lyr in layers:
        for nm in ("W", "b", "gamma", "beta"):
            a = lyr[nm]
            if a.ndim == 1:
                a = a.reshape(1, -1)
            wargs.append(a)
            wspecs.append(pl.BlockSpec(a.shape, lambda i: (0, 0)))
    if head is not None:
        for nm in ("W1", "b1", "W2", "b2", "W3", "b3"):
            a = head[nm]
            if a.ndim == 1:
                a = a.reshape(1, -1)
            wargs.append(a)
            wspecs.append(pl.BlockSpec(a.shape, lambda i: (0, 0)))
        out_f = head["W3"].shape[1]
    else:
        out_f = layers[-1]["W"].shape[1]
    fn = pl.pallas_call(
        functools.partial(_mlp_body, nl, head is not None),
        grid=(rows // br,),
        in_specs=[pl.BlockSpec((br, f), lambda i: (i, 0))] + wspecs,
        out_specs=pl.BlockSpec((br, out_f), lambda i: (i, 0)),
        out_shape=jax.ShapeDtypeStruct((rows, out_f), jnp.float32),
    )
    return fn(h, *wargs)


def _knn_interpolate(xf, posc, pos_skip, k=3):
    d2_sg = jax.lax.stop_gradient(
        jnp.sum((pos_skip[:, None, :] - posc[None, :, :]) ** 2, axis=-1))
    _, idx = jax.lax.approx_max_k(-d2_sg, k, recall_target=1.0)
    diff = pos_skip[:, None, :] - posc[idx]
    d2 = jnp.sum(diff ** 2, axis=-1)
    w = 1.0 / jnp.maximum(d2, 1e-16)
    w = w / jnp.sum(w, axis=1, keepdims=True)
    return jnp.sum(xf[idx] * w[:, :, None], axis=1)


def kernel(x, pos, batch, params):
    x1, pos1 = _sa_module(x, pos, 0.25, [0.05, 0.4], params["sa1"], bm=32)
    x2, pos2 = _sa_module(x1, pos1, 0.25, [0.2, 0.8], params["sa2"], bm=32)
    x3, pos3 = _sa_module(x2, pos2, 0.25, [0.4, 1.6], params["sa3"], bm=32)

    xi3 = _knn_interpolate(x3, pos3, pos2)
    f3 = _mlp_rows(jnp.concatenate([xi3, x2], axis=1), params["fp3"], br=256)
    xi2 = _knn_interpolate(f3, pos2, pos1)
    f2 = _mlp_rows(jnp.concatenate([xi2, x1], axis=1), params["fp2"], br=512)
    xi1 = _knn_interpolate(f2, pos1, pos)
    f1 = _mlp_rows(jnp.concatenate([xi1, x], axis=1), params["fp1"], br=1024)

    return _mlp_rows(f1, [], br=1024, head=params["cls"])


# speedup vs baseline: 5.8145x; 3.1707x over previous
import functools

import jax
import jax.numpy as jnp
import numpy as np
from jax import lax
from jax.experimental import pallas as pl
from jax.experimental.pallas import tpu as pltpu
from jax.experimental.pallas import tpu_sc as plsc

_EPS_BN = 1e-5
_INV = np.float32(1.0) / np.sqrt(np.float32(1.0 + _EPS_BN))

_L = 16
_NB = 272


def _lane_gather(vec, idx):
    return lax.gather(
        vec, idx[:, None],
        dimension_numbers=lax.GatherDimensionNumbers(
            offset_dims=(), collapsed_slice_dims=(0,), start_index_map=(0,)),
        slice_sizes=(1,),
        mode=lax.GatherScatterMode.PROMISE_IN_BOUNDS)


def _ballq_tec(n, k, qpt, *refs):
    (px_h, py_h, pz_h, yx_h, yy_h, yz_h, out_h,
     px_v, py_v, pz_v, yx_v, yy_v, yz_v, bits_v, hist_v, row_v) = refs
    nvec = n // _L
    wid = lax.axis_index("s") * 2 + lax.axis_index("c")

    pltpu.sync_copy(px_h, px_v)
    pltpu.sync_copy(py_h, py_v)
    pltpu.sync_copy(pz_h, pz_v)
    pltpu.sync_copy(yx_h, yx_v)
    pltpu.sync_copy(yy_h, yy_v)
    pltpu.sync_copy(yz_h, yz_v)

    lane = lax.iota(jnp.int32, _L)
    ones = jnp.full((_L,), 1, jnp.int32)

    def clear_hist(j, c):
        hist_v[pl.ds(j * _L, _L)] = jnp.zeros((_L,), jnp.int32)
        return c

    def scan_hist(k_rem):
        def sj(j, st):
            found, bsel, below, run = st
            def sl(l, a):
                return a + hist_v[pl.ds(l * _NB + j * _L, _L)]
            acc = lax.fori_loop(0, _L, sl, jnp.zeros((_L,), jnp.int32))
            tot = jnp.sum(acc)
            cum = plsc.cumsum(acc) + run
            hit = cum > k_rem
            nhit = jnp.sum(hit.astype(jnp.int32))
            ffs = plsc.all_reduce_ffs(hit)
            excl = cum - acc
            below_here = jnp.sum(jnp.where(lane == ffs, excl, 0))
            bin_here = j * _L + jnp.max(ffs)
            take = (found == 0) & (nhit > 0)
            bsel = jnp.where(take, bin_here, bsel)
            below = jnp.where(take, below_here, below)
            found = jnp.where(nhit > 0, 1, found)
            return (found, bsel, below, run + tot)
        z = jnp.int32(0)
        _, bsel, below, _ = lax.fori_loop(0, _NB // _L, sj, (z, z, z, z))
        return bsel, below

    def hist_pass(shift, mask, pshift, prefix):
        lax.fori_loop(0, _NB, clear_hist, 0)
        def pi(i, c):
            b = bits_v[pl.ds(i * _L, _L)]
            binv = (b >> shift) & mask
            m = (b >> pshift) == prefix
            plsc.addupdate_scatter(hist_v, [lane * _NB + binv], ones, mask=m)
            return c
        lax.fori_loop(0, nvec, pi, 0)

    def per_query(lq, carry):
        q = wid * qpt + lq
        qbase = (q // _L) * _L
        qoff = jnp.full((_L,), q - qbase, jnp.int32)
        yx = _lane_gather(yx_v[pl.ds(qbase, _L)], qoff)
        yy = _lane_gather(yy_v[pl.ds(qbase, _L)], qoff)
        yz = _lane_gather(yz_v[pl.ds(qbase, _L)], qoff)

        lax.fori_loop(0, _NB, clear_hist, 0)
        def p1(i, c):
            dx = px_v[pl.ds(i * _L, _L)] - yx
            dy = py_v[pl.ds(i * _L, _L)] - yy
            dz = pz_v[pl.ds(i * _L, _L)] - yz
            d2 = dx * dx + dy * dy + dz * dz
            b = lax.bitcast_convert_type(d2, jnp.int32)
            bits_v[pl.ds(i * _L, _L)] = b
            plsc.addupdate_scatter(hist_v, [lane * _NB + (b >> 22)], ones)
            return c
        lax.fori_loop(0, nvec, p1, 0)

        k0 = jnp.int32(k - 1)
        b1, below1 = scan_hist(k0)
        k1 = k0 - below1

        hist_pass(14, 0xFF, 22, b1)
        b2, below2 = scan_hist(k1)
        k2 = k1 - below2
        pre2 = (b1 << 8) | b2

        hist_pass(6, 0xFF, 14, pre2)
        b3, below3 = scan_hist(k2)
        k3 = k2 - below3
        pre3 = (pre2 << 8) | b3

        hist_pass(0, 0x3F, 6, pre3)
        b4, below4 = scan_hist(k3)

        t = (pre3 << 6) | b4
        count_lt = below1 + below2 + below3 + below4

        def fp(i, st):
            lt_base, eq_base = st
            b = bits_v[pl.ds(i * _L, _L)]
            lt = b < t
            eq = b == t
            lt_i = lt.astype(jnp.int32)
            eq_i = eq.astype(jnp.int32)
            pos_lt = lt_base + plsc.cumsum(lt_i) - 1
            pos_eq = eq_base + plsc.cumsum(eq_i) - 1
            idx_v = i * _L + lane
            plsc.store_scatter(
                row_v, [jnp.minimum(pos_lt, k - 1)], idx_v, mask=lt)
            eqm = eq & (pos_eq < k)
            plsc.store_scatter(
                row_v, [jnp.minimum(pos_eq, k - 1)], idx_v, mask=eqm)
            return (lt_base + jnp.sum(lt_i), eq_base + jnp.sum(eq_i))
        lax.fori_loop(0, nvec, fp, (jnp.int32(0), count_lt))

        pltpu.sync_copy(row_v, out_h.at[q])
        return carry

    lax.fori_loop(0, qpt, per_query, 0)


def _ballq_sc(y_pos, pos, k):
    m = y_pos.shape[0]
    n = pos.shape[0]
    qpt = m // 32
    mesh = plsc.VectorSubcoreMesh(core_axis_name="c", subcore_axis_name="s")
    fn = functools.partial(
        pl.kernel,
        mesh=mesh,
        compiler_params=pltpu.CompilerParams(needs_layout_passes=False),
        out_type=jax.ShapeDtypeStruct((m, k), jnp.int32),
        scratch_types=[
            pltpu.VMEM((n,), jnp.float32),
            pltpu.VMEM((n,), jnp.float32),
            pltpu.VMEM((n,), jnp.float32),
            pltpu.VMEM((m,), jnp.float32),
            pltpu.VMEM((m,), jnp.float32),
            pltpu.VMEM((m,), jnp.float32),
            pltpu.VMEM((n,), jnp.int32),
            pltpu.VMEM((_NB * _L,), jnp.int32),
            pltpu.VMEM((k,), jnp.int32),
        ],
    )(functools.partial(_ballq_tec, n, k, qpt))
    return fn(pos[:, 0], pos[:, 1], pos[:, 2],
              y_pos[:, 0], y_pos[:, 1], y_pos[:, 2])


def _fps_body(m, px_ref, py_ref, pz_ref, out_ref):
    px = px_ref[...]
    py = py_ref[...]
    pz = pz_ref[...]
    r = px.shape[0]
    row = jax.lax.broadcasted_iota(jnp.int32, (r, 128), 0)
    colv = jax.lax.broadcasted_iota(jnp.int32, (r, 128), 1)
    flat = row * 128 + colv
    out_ref[...] = jnp.zeros(out_ref.shape, jnp.int32)
    dists0 = jnp.full((r, 128), jnp.inf, jnp.float32)

    def body(i, carry):
        dists, last = carry
        sel = flat == last
        lx = jnp.sum(jnp.where(sel, px, 0.0))
        ly = jnp.sum(jnp.where(sel, py, 0.0))
        lz = jnp.sum(jnp.where(sel, pz, 0.0))
        dxx = px - lx
        dyy = py - ly
        dzz = pz - lz
        d = dxx * dxx + dyy * dyy + dzz * dzz
        dists = jnp.minimum(dists, d)
        mx = jnp.max(dists)
        idx = jnp.min(jnp.where(dists == mx, flat, jnp.int32(2 ** 30)))
        out_ref[pl.ds(i, 1), :] = jnp.reshape(idx, (1, 1))
        return (dists, idx)

    jax.lax.fori_loop(1, m, body, (dists0, jnp.int32(0)))


def _fps_idx(pos, num_samples):
    n = pos.shape[0]
    r = n // 128
    px = pos[:, 0].reshape(r, 128)
    py = pos[:, 1].reshape(r, 128)
    pz = pos[:, 2].reshape(r, 128)
    out = pl.pallas_call(
        functools.partial(_fps_body, num_samples),
        out_shape=jax.ShapeDtypeStruct((num_samples, 1), jnp.int32),
    )(px, py, pz)
    return out.reshape(num_samples)


def _sa_body(nl, kk, r2s, fouts, *refs):
    h_ref, d2_ref = refs[0], refs[1]
    wrefs = refs[2:-1]
    out_ref = refs[-1]
    h0 = h_ref[...]
    d2col = d2_ref[...]
    bm = d2col.shape[0] // kk
    col = 0
    for bi, r2 in enumerate(r2s):
        h = h0
        base = bi * nl * 4
        for li in range(nl):
            w = wrefs[base + li * 4][...]
            b = wrefs[base + li * 4 + 1][...]
            g = wrefs[base + li * 4 + 2][...]
            be = wrefs[base + li * 4 + 3][...]
            h = jnp.maximum(
                jnp.dot(h, w, preferred_element_type=jnp.float32) + b, 0.0)
            h = g * (h * _INV) + be
        fo = fouts[bi]
        penalty = jnp.where(d2col <= r2, 0.0, -jnp.inf)
        h = h + penalty
        o = jnp.max(h.reshape(bm, kk, fo), axis=1)
        o = jnp.where(jnp.isfinite(o), o, 0.0)
        out_ref[:, col:col + fo] = o
        col += fo


def _sa_conv(h_in, d2k, r_list, conv_params, bm):
    m, kk, f = h_in.shape
    h_flat = h_in.reshape(m * kk, f)
    d2col = d2k.reshape(m * kk, 1)
    nl = len(conv_params[0])
    fouts = tuple(int(layers[-1]["W"].shape[1]) for layers in conv_params)
    r2s = tuple(np.float32(r * r) for r in r_list)
    wargs, wspecs = [], []
    for layers in conv_params:
        for lyr in layers:
            for nm in ("W", "b", "gamma", "beta"):
                a = lyr[nm]
                if a.ndim == 1:
                    a = a.reshape(1, -1)
                wargs.append(a)
                wspecs.append(pl.BlockSpec(a.shape, lambda i: (0, 0)))
    out_f = sum(fouts)
    grid = (m // bm,)
    fn = pl.pallas_call(
        functools.partial(_sa_body, nl, kk, r2s, fouts),
        grid=grid,
        in_specs=[
            pl.BlockSpec((bm * kk, f), lambda i: (i, 0)),
            pl.BlockSpec((bm * kk, 1), lambda i: (i, 0)),
        ] + wspecs,
        out_specs=pl.BlockSpec((bm, out_f), lambda i: (i, 0)),
        out_shape=jax.ShapeDtypeStruct((m, out_f), jnp.float32),
    )
    return fn(h_flat, d2col, *wargs)


def _sa_module(x, pos, ratio, r_list, conv_params, bm, max_nbrs=128):
    n = pos.shape[0]
    m = int(round(ratio * n))
    idx = _fps_idx(pos, m)
    y_pos = pos[idx]
    nbr = _ballq_sc(y_pos, pos, max_nbrs)
    x_j = x[nbr]
    rel = pos[nbr] - y_pos[:, None, :]
    d2k = jnp.sum(rel ** 2, axis=-1)
    h_in = jnp.concatenate([x_j, rel], axis=-1)
    return _sa_conv(h_in, d2k, r_list, conv_params, bm), y_pos


def _mlp_body(nl, with_head, *refs):
    h_ref = refs[0]
    wrefs = refs[1:-1]
    out_ref = refs[-1]
    h = h_ref[...]
    for li in range(nl):
        w = wrefs[li * 4][...]
        b = wrefs[li * 4 + 1][...]
        g = wrefs[li * 4 + 2][...]
        be = wrefs[li * 4 + 3][...]
        h = jnp.maximum(
            jnp.dot(h, w, preferred_element_type=jnp.float32) + b, 0.0)
        h = g * (h * _INV) + be
    if with_head:
        base = nl * 4
        w1, b1 = wrefs[base][...], wrefs[base + 1][...]
        w2, b2 = wrefs[base + 2][...], wrefs[base + 3][...]
        w3, b3 = wrefs[base + 4][...], wrefs[base + 5][...]
        h = jnp.maximum(jnp.dot(h, w1, preferred_element_type=jnp.float32) + b1, 0.0)
        h = jnp.maximum(jnp.dot(h, w2, preferred_element_type=jnp.float32) + b2, 0.0)
        h = jnp.dot(h, w3, preferred_element_type=jnp.float32) + b3
        mx = jnp.max(h, axis=-1, keepdims=True)
        sh = h - jax.lax.stop_gradient(mx)
        h = sh - jnp.log(jnp.sum(jnp.exp(sh), axis=-1, keepdims=True))
    out_ref[...] = h


def _mlp_rows(h, layers, br, head=None):
    rows, f = h.shape
    nl = len(layers)
    wargs, wspecs = [], []
    for lyr in layers:
        for nm in ("W", "b", "gamma", "beta"):
            a = lyr[nm]
            if a.ndim == 1:
                a = a.reshape(1, -1)
            wargs.append(a)
            wspecs.append(pl.BlockSpec(a.shape, lambda i: (0, 0)))
    if head is not None:
        for nm in ("W1", "b1", "W2", "b2", "W3", "b3"):
            a = head[nm]
            if a.ndim == 1:
                a = a.reshape(1, -1)
            wargs.append(a)
            wspecs.append(pl.BlockSpec(a.shape, lambda i: (0, 0)))
        out_f = head["W3"].shape[1]
    else:
        out_f = layers[-1]["W"].shape[1]
    fn = pl.pallas_call(
        functools.partial(_mlp_body, nl, head is not None),
        grid=(rows // br,),
        in_specs=[pl.BlockSpec((br, f), lambda i: (i, 0))] + wspecs,
        out_specs=pl.BlockSpec((br, out_f), lambda i: (i, 0)),
        out_shape=jax.ShapeDtypeStruct((rows, out_f), jnp.float32),
    )
    return fn(h, *wargs)


def _knn_interpolate(xf, posc, pos_skip, k=3):
    d2_sg = jax.lax.stop_gradient(
        jnp.sum((pos_skip[:, None, :] - posc[None, :, :]) ** 2, axis=-1))
    _, idx = jax.lax.approx_max_k(-d2_sg, k, recall_target=1.0)
    diff = pos_skip[:, None, :] - posc[idx]
    d2 = jnp.sum(diff ** 2, axis=-1)
    w = 1.0 / jnp.maximum(d2, 1e-16)
    w = w / jnp.sum(w, axis=1, keepdims=True)
    return jnp.sum(xf[idx] * w[:, :, None], axis=1)


def kernel(x, pos, batch, params):
    x1, pos1 = _sa_module(x, pos, 0.25, [0.05, 0.4], params["sa1"], bm=32)
    x2, pos2 = _sa_module(x1, pos1, 0.25, [0.2, 0.8], params["sa2"], bm=32)
    x3, pos3 = _sa_module(x2, pos2, 0.25, [0.4, 1.6], params["sa3"], bm=32)

    xi3 = _knn_interpolate(x3, pos3, pos2)
    f3 = _mlp_rows(jnp.concatenate([xi3, x2], axis=1), params["fp3"], br=256)
    xi2 = _knn_interpolate(f3, pos2, pos1)
    f2 = _mlp_rows(jnp.concatenate([xi2, x1], axis=1), params["fp2"], br=512)
    xi1 = _knn_interpolate(f2, pos1, pos)
    f1 = _mlp_rows(jnp.concatenate([xi1, x], axis=1), params["fp1"], br=1024)

    return _mlp_rows(f1, [], br=1024, head=params["cls"])
